# baseline (speedup 1.0000x reference)
import dataclasses
import functools

import jax
import jax.numpy as jnp
from jax import lax
from jax.experimental import pallas as pl
from jax.experimental.pallas import tpu as pltpu
from jax.experimental.pallas import tpu_sc as plsc

N = 10000
E = 160000
D = 128
DD = 64

NC = 2
NS = 16
NW = NC * NS
GB = 128
EC = E // 2
CBLK = EC // GB
GW_FULL = CBLK // NW + 1
SW_FULL = CBLK // NS + 1

NROWS = 624
NREM = N - 15 * NROWS - NROWS
ZR = 16

NB = 1000
EB = 1000


def _bdot(a, b):
    return jnp.dot(a.astype(jnp.bfloat16), b.astype(jnp.bfloat16),
                   preferred_element_type=jnp.float32)


def _node_body(nf_ref, wsk_ref, wup_ref, wdn_ref, sc_ref, pack_ref):
    nf = nf_ref[...]
    sc_ref[...] = _bdot(nf, wsk_ref[...])
    up = _bdot(nf, wup_ref[...])
    dn = _bdot(nf, wdn_ref[...])
    pad = jnp.zeros((nf.shape[0], DD), jnp.float32)
    pack_ref[...] = jnp.concatenate([up, dn, pad], axis=1).astype(jnp.bfloat16)


def _node_call(node_feats, w_skip, w_up, w_down):
    return pl.pallas_call(
        _node_body,
        grid=(N // NB,),
        in_specs=[
            pl.BlockSpec((NB, D), lambda i: (i, 0)),
            pl.BlockSpec((D, D), lambda i: (0, 0)),
            pl.BlockSpec((D, D), lambda i: (0, 0)),
            pl.BlockSpec((D, DD), lambda i: (0, 0)),
        ],
        out_specs=[
            pl.BlockSpec((NB, D), lambda i: (i, 0)),
            pl.BlockSpec((NB, 2 * D), lambda i: (i, 0)),
        ],
        out_shape=[
            jax.ShapeDtypeStruct((N, D), jnp.float32),
            jax.ShapeDtypeStruct((N, 2 * D), jnp.bfloat16),
        ],
    )(node_feats, w_skip, w_up, w_down)


def _gather_body(tab_hbm, snd_hbm, rcv_hbm, gs_hbm, gr_hbm,
                 idx_s, idx_r, rows_sa, rows_sb, rows_ra, rows_rb,
                 sem_sa, sem_sb, sem_ra, sem_rb):
    cid = lax.axis_index("c")
    sid = lax.axis_index("s")
    wid = sid * NC + cid
    nblk = jnp.where(wid < CBLK % NW, CBLK // NW + 1, CBLK // NW)

    @pl.loop(0, nblk)
    def _(b):
        off = (wid + b * NW) * GB
        pltpu.sync_copy(snd_hbm.at[pl.ds(off, GB)], idx_s.at[b])
        pltpu.sync_copy(rcv_hbm.at[pl.ds(off, GB)], idx_r.at[b])

    def start(b, rows_s, rows_r, sem_s, sem_r):
        pltpu.async_copy(tab_hbm.at[idx_s.at[b]], rows_s, sem_s)
        pltpu.async_copy(tab_hbm.at[idx_r.at[b]], rows_r, sem_r)

    def drain(b, rows_s, rows_r, sem_s, sem_r):
        pltpu.make_async_copy(tab_hbm.at[idx_s.at[0]], rows_s, sem_s).wait()
        pltpu.make_async_copy(tab_hbm.at[idx_r.at[0]], rows_r, sem_r).wait()
        off = (wid + b * NW) * GB
        pltpu.sync_copy(rows_s, gs_hbm.at[pl.ds(off, GB)])
        pltpu.sync_copy(rows_r, gr_hbm.at[pl.ds(off, GB)])

    start(0, rows_sa, rows_ra, sem_sa, sem_ra)

    @pl.loop(0, (GW_FULL + 1) // 2)
    def _(i):
        b = 2 * i

        @pl.when(b + 1 < nblk)
        def _():
            start(b + 1, rows_sb, rows_rb, sem_sb, sem_rb)

        @pl.when(b < nblk)
        def _():
            drain(b, rows_sa, rows_ra, sem_sa, sem_ra)

        @pl.when(b + 2 < nblk)
        def _():
            start(b + 2, rows_sa, rows_ra, sem_sa, sem_ra)

        @pl.when(b + 1 < nblk)
        def _():
            drain(b + 1, rows_sb, rows_rb, sem_sb, sem_rb)


def _gather_call(tab, sender, receiver):
    mesh = plsc.VectorSubcoreMesh(core_axis_name="c", subcore_axis_name="s")
    f = pl.kernel(
        _gather_body,
        out_type=[
            jax.ShapeDtypeStruct((EC, D), jnp.float32),
            jax.ShapeDtypeStruct((EC, D), jnp.float32),
        ],
        mesh=mesh,
        scratch_types=[
            pltpu.VMEM((GW_FULL, GB), jnp.int32),
            pltpu.VMEM((GW_FULL, GB), jnp.int32),
            pltpu.VMEM((GB, D), jnp.float32),
            pltpu.VMEM((GB, D), jnp.float32),
            pltpu.VMEM((GB, D), jnp.float32),
            pltpu.VMEM((GB, D), jnp.float32),
            pltpu.SemaphoreType.DMA,
            pltpu.SemaphoreType.DMA,
            pltpu.SemaphoreType.DMA,
            pltpu.SemaphoreType.DMA,
        ],
    )
    return f(tab, sender, receiver)


def _halves(ref):
    xi = jax.lax.bitcast_convert_type(ref[...], jnp.uint32)
    lo = jax.lax.bitcast_convert_type(xi << 16, jnp.float32)
    hi = jax.lax.bitcast_convert_type(xi & jnp.uint32(0xFFFF0000), jnp.float32)
    return lo, hi


def _mlp_body(ef_ref, ea_ref, gs_ref, gr_ref,
              wm0_ref, wm1_ref, wm2_ref, wm3_ref, y_ref):
    slo, shi = _halves(gs_ref)
    rlo, rhi = _halves(gr_ref)
    xs = jnp.concatenate([slo[:, :D // 2], shi[:, :D // 2]], axis=1)
    sdn = jnp.concatenate([slo[:, D // 2:D // 2 + DD // 2],
                           shi[:, D // 2:D // 2 + DD // 2]], axis=1)
    rdn = jnp.concatenate([rlo[:, D // 2:D // 2 + DD // 2],
                           rhi[:, D // 2:D // 2 + DD // 2]], axis=1)
    aug = jnp.concatenate([ef_ref[...].astype(jnp.bfloat16),
                           sdn.astype(jnp.bfloat16),
                           rdn.astype(jnp.bfloat16)], axis=1)
    h = jax.nn.silu(_bdot(aug, wm0_ref[...]))
    h = jax.nn.silu(_bdot(h, wm1_ref[...]))
    h = jax.nn.silu(_bdot(h, wm2_ref[...]))
    tpw = _bdot(h, wm3_ref[...])
    ea = ea_ref[...]
    y0 = xs * tpw[:, :D]
    y1 = xs * tpw[:, D:2 * D]
    y2 = xs * tpw[:, 2 * D:]
    comps = [y0 * ea[:, 0:1]]
    comps += [y1 * ea[:, k:k + 1] for k in range(1, 4)]
    comps += [y2 * ea[:, k:k + 1] for k in range(4, 9)]
    y_ref[...] = jnp.stack(comps)


def _mlp_call(edge_feats, edge_attrs, gs, gr, wm0, wm1, wm2, wm3, chunk):
    co = chunk * (EC // EB)
    return pl.pallas_call(
        _mlp_body,
        grid=(EC // EB,),
        in_specs=[
            pl.BlockSpec((EB, 8), lambda i: (i + co, 0)),
            pl.BlockSpec((EB, 9), lambda i: (i + co, 0)),
            pl.BlockSpec((EB, D), lambda i: (i, 0)),
            pl.BlockSpec((EB, D), lambda i: (i, 0)),
            pl.BlockSpec((8 + 2 * DD, 256), lambda i: (0, 0)),
            pl.BlockSpec((256, 256), lambda i: (0, 0)),
            pl.BlockSpec((256, 256), lambda i: (0, 0)),
            pl.BlockSpec((256, 3 * D), lambda i: (0, 0)),
        ],
        out_specs=pl.BlockSpec((9, EB, D), lambda i: (0, i, 0)),
        out_shape=jax.ShapeDtypeStruct((9, EC, D), jnp.float32),
    )(edge_feats, edge_attrs, gs, gr, wm0, wm1, wm2, wm3)


def _scatter_body(y0_hbm, y1_hbm, rcv_hbm, parts_hbm,
                  acc, ybuf_a, ybuf_b, idxall, zbuf, sem_a, sem_b):
    cid = lax.axis_index("c")
    sid = lax.axis_index("s")
    nblk = jnp.where(sid < CBLK % NS, CBLK // NS + 1, CBLK // NS)
    zero16 = jnp.zeros((16,), jnp.float32)

    @pl.loop(0, ZR)
    def _(r):
        for c in range(D // 16):
            zbuf[r, pl.ds(c * 16, 16)] = zero16

    @pl.loop(0, nblk)
    def _(b):
        off = cid * EC + (sid + b * NS) * GB
        pltpu.sync_copy(rcv_hbm.at[pl.ds(off, GB)], idxall.at[b])

    def passes(y_hbm):
        def start(j, b, buf, sem):
            pltpu.async_copy(y_hbm.at[j, pl.ds((sid + b * NS) * GB, GB)],
                             buf, sem)

        def wait(j, buf, sem):
            pltpu.make_async_copy(y_hbm.at[j, pl.ds(0, GB)], buf, sem).wait()

        def scat(b, buf):
            pltpu.sync_copy(buf, acc.at[idxall.at[b]], add=True)

        @pl.loop(0, 9)
        def _(j):
            @pl.loop(0, NROWS // ZR)
            def _(k):
                pltpu.sync_copy(zbuf, acc.at[pl.ds(sid * NROWS + k * ZR, ZR)])

            @pl.when(sid == NS - 1)
            def _():
                pltpu.sync_copy(zbuf, acc.at[pl.ds(N - NREM, NREM)])
            plsc.subcore_barrier()

            start(j, 0, ybuf_a, sem_a)

            @pl.loop(0, (SW_FULL + 1) // 2)
            def _(i):
                b = 2 * i

                @pl.when(b + 1 < nblk)
                def _():
                    start(j, b + 1, ybuf_b, sem_b)

                @pl.when(b < nblk)
                def _():
                    wait(j, ybuf_a, sem_a)
                    scat(b, ybuf_a)

                @pl.when(b + 2 < nblk)
                def _():
                    start(j, b + 2, ybuf_a, sem_a)

                @pl.when(b + 1 < nblk)
                def _():
                    wait(j, ybuf_b, sem_b)
                    scat(b + 1, ybuf_b)

            plsc.subcore_barrier()
            pltpu.sync_copy(acc.at[pl.ds(sid * NROWS, NROWS)],
                            parts_hbm.at[cid, j, pl.ds(sid * NROWS, NROWS)])

            @pl.when(sid == NS - 1)
            def _():
                pltpu.sync_copy(acc.at[pl.ds(N - NREM, NREM)],
                                parts_hbm.at[cid, j, pl.ds(N - NREM, NREM)])
            plsc.subcore_barrier()

    @pl.when(cid == 0)
    def _():
        passes(y0_hbm)

    @pl.when(cid == 1)
    def _():
        passes(y1_hbm)


def _scatter_call(y0, y1, receiver):
    mesh = plsc.VectorSubcoreMesh(core_axis_name="c", subcore_axis_name="s")
    cp = pltpu.CompilerParams()
    if "needs_layout_passes" in pltpu.CompilerParams.__dataclass_fields__:
        cp = dataclasses.replace(cp, needs_layout_passes=False)
    f = pl.kernel(
        _scatter_body,
        out_type=jax.ShapeDtypeStruct((NC, 9, N, D), jnp.float32),
        mesh=mesh,
        compiler_params=cp,
        scratch_types=[
            pltpu.VMEM_SHARED((N, D), jnp.float32),
            pltpu.VMEM((GB, D), jnp.float32),
            pltpu.VMEM((GB, D), jnp.float32),
            pltpu.VMEM((SW_FULL, GB), jnp.int32),
            pltpu.VMEM((ZR, D), jnp.float32),
            pltpu.SemaphoreType.DMA,
            pltpu.SemaphoreType.DMA,
        ],
    )
    return f(y0, y1, receiver)


def _mix_body3(parts_ref, wl0_ref, wl1_ref, wl2_ref, out_ref):
    inv = jnp.float32(1.0 / 16.0)
    p = parts_ref[...]
    outs = []
    for j in range(9):
        msg = p[0, j] + p[1, j]
        if j == 0:
            w = wl0_ref[...]
        elif j < 4:
            w = wl1_ref[...]
        else:
            w = wl2_ref[...]
        outs.append(_bdot(msg, w) * inv)
    out_ref[...] = jnp.stack(outs, axis=1)


def _mix_call3(parts, wl0, wl1, wl2):
    return pl.pallas_call(
        _mix_body3,
        grid=(N // NB,),
        in_specs=[
            pl.BlockSpec((NC, 9, NB, D), lambda i: (0, 0, i, 0)),
            pl.BlockSpec((D, D), lambda i: (0, 0)),
            pl.BlockSpec((D, D), lambda i: (0, 0)),
            pl.BlockSpec((D, D), lambda i: (0, 0)),
        ],
        out_specs=pl.BlockSpec((NB, 9, D), lambda i: (i, 0, 0)),
        out_shape=jax.ShapeDtypeStruct((N, 9, D), jnp.float32),
    )(parts, wl0, wl1, wl2)


def kernel(node_attrs, node_feats, edge_attrs, edge_feats, edge_index,
           W_up, W_down, Wm0, Wm1, Wm2, Wm3, Wl0, Wl1, Wl2, W_skip):
    sender = edge_index[0]
    receiver = edge_index[1]

    perm = jnp.concatenate([jnp.arange(0, D, 2), jnp.arange(1, D, 2)])
    perm2 = jnp.concatenate([jnp.arange(0, DD, 2), jnp.arange(1, DD, 2)])
    wm0_p = jnp.concatenate([Wm0[:8], Wm0[8:8 + DD][perm2],
                             Wm0[8 + DD:8 + 2 * DD][perm2]], axis=0)
    wm3_p = Wm3.reshape(256, 3, D)[:, :, perm].reshape(256, 3 * D)
    wl0_p = Wl0[perm]
    wl1_p = Wl1[perm]
    wl2_p = Wl2[perm]

    sc, pack = _node_call(node_feats, W_skip, W_up, W_down)
    tab = jax.lax.bitcast_convert_type(pack.reshape(N, D, 2), jnp.float32)
    gs0, gr0 = _gather_call(tab, sender[:EC], receiver[:EC])
    gs1, gr1 = _gather_call(tab, sender[EC:], receiver[EC:])
    y0 = _mlp_call(edge_feats, edge_attrs, gs0, gr0, wm0_p, Wm1, Wm2, wm3_p, 0)
    y1 = _mlp_call(edge_feats, edge_attrs, gs1, gr1, wm0_p, Wm1, Wm2, wm3_p, 1)
    parts = _scatter_call(y0, y1, receiver)
    mixed = _mix_call3(parts, wl0_p, wl1_p, wl2_p)
    reshaped = jnp.transpose(mixed, (0, 2, 1))
    return (reshaped, sc)


# speedup vs baseline: 1.0468x; 1.0468x over previous
import dataclasses
import functools

import jax
import jax.numpy as jnp
from jax import lax
from jax.experimental import pallas as pl
from jax.experimental.pallas import tpu as pltpu
from jax.experimental.pallas import tpu_sc as plsc

N = 10000
E = 160000
D = 128
DD = 64

NC = 2
NS = 16
NW = NC * NS
GB = 128
EC = E // 2
CBLK = EC // GB
GW_FULL = CBLK // NW + 1
SW_FULL = CBLK // NS + 1

NROWS = 624
NREM = N - 15 * NROWS - NROWS
ZR = 16

NB = 1000
EB = 1000


def _bdot(a, b):
    return jnp.dot(a.astype(jnp.bfloat16), b.astype(jnp.bfloat16),
                   preferred_element_type=jnp.float32)


def _node_body(nf_ref, wsk_ref, wup_ref, wdn_ref, sc_ref, xcat_ref, xdn_ref):
    nf = nf_ref[...]
    sc_ref[...] = _bdot(nf, wsk_ref[...])
    up = _bdot(nf, wup_ref[...])
    dn = _bdot(nf, wdn_ref[...])
    pad = jnp.zeros((nf.shape[0], DD), jnp.float32)
    xcat_ref[...] = jnp.concatenate([up, dn, pad], axis=1)
    xdn_ref[...] = jnp.concatenate([dn, pad], axis=1)


def _node_call(node_feats, w_skip, w_up, w_down):
    return pl.pallas_call(
        _node_body,
        grid=(N // NB,),
        in_specs=[
            pl.BlockSpec((NB, D), lambda i: (i, 0)),
            pl.BlockSpec((D, D), lambda i: (0, 0)),
            pl.BlockSpec((D, D), lambda i: (0, 0)),
            pl.BlockSpec((D, DD), lambda i: (0, 0)),
        ],
        out_specs=[
            pl.BlockSpec((NB, D), lambda i: (i, 0)),
            pl.BlockSpec((NB, 2 * D), lambda i: (i, 0)),
            pl.BlockSpec((NB, D), lambda i: (i, 0)),
        ],
        out_shape=[
            jax.ShapeDtypeStruct((N, D), jnp.float32),
            jax.ShapeDtypeStruct((N, 2 * D), jnp.float32),
            jax.ShapeDtypeStruct((N, D), jnp.float32),
        ],
    )(node_feats, w_skip, w_up, w_down)


def _gather_body(xcat_hbm, xdn_hbm, snd_hbm, rcv_hbm, gs_hbm, gr_hbm,
                 idx_s, idx_r, rows_sa, rows_sb, rows_ra, rows_rb,
                 sem_sa, sem_sb, sem_ra, sem_rb):
    cid = lax.axis_index("c")
    sid = lax.axis_index("s")
    wid = sid * NC + cid
    nblk = jnp.where(wid < CBLK % NW, CBLK // NW + 1, CBLK // NW)

    @pl.loop(0, nblk)
    def _(b):
        off = (wid + b * NW) * GB
        pltpu.sync_copy(snd_hbm.at[pl.ds(off, GB)], idx_s.at[b])
        pltpu.sync_copy(rcv_hbm.at[pl.ds(off, GB)], idx_r.at[b])

    def start(b, rows_s, rows_r, sem_s, sem_r):
        pltpu.async_copy(xcat_hbm.at[idx_s.at[b]], rows_s, sem_s)
        pltpu.async_copy(xdn_hbm.at[idx_r.at[b]], rows_r, sem_r)

    def drain(b, rows_s, rows_r, sem_s, sem_r):
        pltpu.make_async_copy(xcat_hbm.at[idx_s.at[0]], rows_s, sem_s).wait()
        pltpu.make_async_copy(xdn_hbm.at[idx_r.at[0]], rows_r, sem_r).wait()
        off = (wid + b * NW) * GB
        pltpu.sync_copy(rows_s, gs_hbm.at[pl.ds(off, GB)])
        pltpu.sync_copy(rows_r, gr_hbm.at[pl.ds(off, GB)])

    start(0, rows_sa, rows_ra, sem_sa, sem_ra)

    @pl.loop(0, (GW_FULL + 1) // 2)
    def _(i):
        b = 2 * i

        @pl.when(b + 1 < nblk)
        def _():
            start(b + 1, rows_sb, rows_rb, sem_sb, sem_rb)

        @pl.when(b < nblk)
        def _():
            drain(b, rows_sa, rows_ra, sem_sa, sem_ra)

        @pl.when(b + 2 < nblk)
        def _():
            start(b + 2, rows_sa, rows_ra, sem_sa, sem_ra)

        @pl.when(b + 1 < nblk)
        def _():
            drain(b + 1, rows_sb, rows_rb, sem_sb, sem_rb)


def _gather_call(xcat, xdn, sender, receiver):
    mesh = plsc.VectorSubcoreMesh(core_axis_name="c", subcore_axis_name="s")
    f = pl.kernel(
        _gather_body,
        out_type=[
            jax.ShapeDtypeStruct((EC, 2 * D), jnp.float32),
            jax.ShapeDtypeStruct((EC, D), jnp.float32),
        ],
        mesh=mesh,
        scratch_types=[
            pltpu.VMEM((GW_FULL, GB), jnp.int32),
            pltpu.VMEM((GW_FULL, GB), jnp.int32),
            pltpu.VMEM((GB, 2 * D), jnp.float32),
            pltpu.VMEM((GB, 2 * D), jnp.float32),
            pltpu.VMEM((GB, D), jnp.float32),
            pltpu.VMEM((GB, D), jnp.float32),
            pltpu.SemaphoreType.DMA,
            pltpu.SemaphoreType.DMA,
            pltpu.SemaphoreType.DMA,
            pltpu.SemaphoreType.DMA,
        ],
    )
    return f(xcat, xdn, sender, receiver)


def _mlp_body(ef_ref, ea_ref, gs_ref, gr_ref,
              wm0_ref, wm1_ref, wm2_ref, wm3_ref, y_ref):
    gs = gs_ref[...]
    xs = gs[:, :D]
    aug = jnp.concatenate([ef_ref[...], gs[:, D:D + DD], gr_ref[...][:, :DD]],
                          axis=1)
    h = jax.nn.silu(_bdot(aug, wm0_ref[...]))
    h = jax.nn.silu(_bdot(h, wm1_ref[...]))
    h = jax.nn.silu(_bdot(h, wm2_ref[...]))
    tpw = _bdot(h, wm3_ref[...])
    ea = ea_ref[...]
    y0 = xs * tpw[:, :D]
    y1 = xs * tpw[:, D:2 * D]
    y2 = xs * tpw[:, 2 * D:]
    comps = [y0 * ea[:, 0:1]]
    comps += [y1 * ea[:, k:k + 1] for k in range(1, 4)]
    comps += [y2 * ea[:, k:k + 1] for k in range(4, 9)]
    y_ref[...] = jnp.stack(comps)


def _mlp_call(edge_feats, edge_attrs, gs, gr, wm0, wm1, wm2, wm3, chunk):
    co = chunk * (EC // EB)
    return pl.pallas_call(
        _mlp_body,
        grid=(EC // EB,),
        in_specs=[
            pl.BlockSpec((EB, 8), lambda i: (i + co, 0)),
            pl.BlockSpec((EB, 9), lambda i: (i + co, 0)),
            pl.BlockSpec((EB, 2 * D), lambda i: (i, 0)),
            pl.BlockSpec((EB, D), lambda i: (i, 0)),
            pl.BlockSpec((8 + 2 * DD, 256), lambda i: (0, 0)),
            pl.BlockSpec((256, 256), lambda i: (0, 0)),
            pl.BlockSpec((256, 256), lambda i: (0, 0)),
            pl.BlockSpec((256, 3 * D), lambda i: (0, 0)),
        ],
        out_specs=pl.BlockSpec((9, EB, D), lambda i: (0, i, 0)),
        out_shape=jax.ShapeDtypeStruct((9, EC, D), jnp.float32),
    )(edge_feats, edge_attrs, gs, gr, wm0, wm1, wm2, wm3)


def _scatter_body(y0_hbm, y1_hbm, rcv_hbm, parts_hbm,
                  acc, ybuf_a, ybuf_b, idxall, zbuf, sem_a, sem_b):
    cid = lax.axis_index("c")
    sid = lax.axis_index("s")
    nblk = jnp.where(sid < CBLK % NS, CBLK // NS + 1, CBLK // NS)
    zero16 = jnp.zeros((16,), jnp.float32)

    @pl.loop(0, ZR)
    def _(r):
        for c in range(D // 16):
            zbuf[r, pl.ds(c * 16, 16)] = zero16

    @pl.loop(0, nblk)
    def _(b):
        off = cid * EC + (sid + b * NS) * GB
        pltpu.sync_copy(rcv_hbm.at[pl.ds(off, GB)], idxall.at[b])

    def passes(y_hbm):
        def start(j, b, buf, sem):
            pltpu.async_copy(y_hbm.at[j, pl.ds((sid + b * NS) * GB, GB)],
                             buf, sem)

        def wait(j, buf, sem):
            pltpu.make_async_copy(y_hbm.at[j, pl.ds(0, GB)], buf, sem).wait()

        def scat(b, buf):
            pltpu.sync_copy(buf, acc.at[idxall.at[b]], add=True)

        @pl.loop(0, 9)
        def _(j):
            @pl.loop(0, NROWS // ZR)
            def _(k):
                pltpu.sync_copy(zbuf, acc.at[pl.ds(sid * NROWS + k * ZR, ZR)])

            @pl.when(sid == NS - 1)
            def _():
                pltpu.sync_copy(zbuf, acc.at[pl.ds(N - NREM, NREM)])
            plsc.subcore_barrier()

            start(j, 0, ybuf_a, sem_a)

            @pl.loop(0, (SW_FULL + 1) // 2)
            def _(i):
                b = 2 * i

                @pl.when(b + 1 < nblk)
                def _():
                    start(j, b + 1, ybuf_b, sem_b)

                @pl.when(b < nblk)
                def _():
                    wait(j, ybuf_a, sem_a)
                    scat(b, ybuf_a)

                @pl.when(b + 2 < nblk)
                def _():
                    start(j, b + 2, ybuf_a, sem_a)

                @pl.when(b + 1 < nblk)
                def _():
                    wait(j, ybuf_b, sem_b)
                    scat(b + 1, ybuf_b)

            plsc.subcore_barrier()
            pltpu.sync_copy(acc.at[pl.ds(sid * NROWS, NROWS)],
                            parts_hbm.at[cid, j, pl.ds(sid * NROWS, NROWS)])

            @pl.when(sid == NS - 1)
            def _():
                pltpu.sync_copy(acc.at[pl.ds(N - NREM, NREM)],
                                parts_hbm.at[cid, j, pl.ds(N - NREM, NREM)])
            plsc.subcore_barrier()

    @pl.when(cid == 0)
    def _():
        passes(y0_hbm)

    @pl.when(cid == 1)
    def _():
        passes(y1_hbm)


def _scatter_call(y0, y1, receiver):
    mesh = plsc.VectorSubcoreMesh(core_axis_name="c", subcore_axis_name="s")
    cp = pltpu.CompilerParams()
    if "needs_layout_passes" in pltpu.CompilerParams.__dataclass_fields__:
        cp = dataclasses.replace(cp, needs_layout_passes=False)
    f = pl.kernel(
        _scatter_body,
        out_type=jax.ShapeDtypeStruct((NC, 9, N, D), jnp.float32),
        mesh=mesh,
        compiler_params=cp,
        scratch_types=[
            pltpu.VMEM_SHARED((N, D), jnp.float32),
            pltpu.VMEM((GB, D), jnp.float32),
            pltpu.VMEM((GB, D), jnp.float32),
            pltpu.VMEM((SW_FULL, GB), jnp.int32),
            pltpu.VMEM((ZR, D), jnp.float32),
            pltpu.SemaphoreType.DMA,
            pltpu.SemaphoreType.DMA,
        ],
    )
    return f(y0, y1, receiver)


def _mix_body3(parts_ref, wl0_ref, wl1_ref, wl2_ref, out_ref):
    inv = jnp.float32(1.0 / 16.0)
    p = parts_ref[...]
    outs = []
    for j in range(9):
        msg = p[0, j] + p[1, j]
        if j == 0:
            w = wl0_ref[...]
        elif j < 4:
            w = wl1_ref[...]
        else:
            w = wl2_ref[...]
        outs.append(_bdot(msg, w) * inv)
    out_ref[...] = jnp.stack(outs, axis=1)


def _mix_call3(parts, wl0, wl1, wl2):
    return pl.pallas_call(
        _mix_body3,
        grid=(N // NB,),
        in_specs=[
            pl.BlockSpec((NC, 9, NB, D), lambda i: (0, 0, i, 0)),
            pl.BlockSpec((D, D), lambda i: (0, 0)),
            pl.BlockSpec((D, D), lambda i: (0, 0)),
            pl.BlockSpec((D, D), lambda i: (0, 0)),
        ],
        out_specs=pl.BlockSpec((NB, 9, D), lambda i: (i, 0, 0)),
        out_shape=jax.ShapeDtypeStruct((N, 9, D), jnp.float32),
    )(parts, wl0, wl1, wl2)


def kernel(node_attrs, node_feats, edge_attrs, edge_feats, edge_index,
           W_up, W_down, Wm0, Wm1, Wm2, Wm3, Wl0, Wl1, Wl2, W_skip):
    sender = edge_index[0]
    receiver = edge_index[1]

    sc, xcat, xdn = _node_call(node_feats, W_skip, W_up, W_down)
    gs0, gr0 = _gather_call(xcat, xdn, sender[:EC], receiver[:EC])
    gs1, gr1 = _gather_call(xcat, xdn, sender[EC:], receiver[EC:])
    y0 = _mlp_call(edge_feats, edge_attrs, gs0, gr0, Wm0, Wm1, Wm2, Wm3, 0)
    y1 = _mlp_call(edge_feats, edge_attrs, gs1, gr1, Wm0, Wm1, Wm2, Wm3, 1)
    parts = _scatter_call(y0, y1, receiver)
    mixed = _mix_call3(parts, Wl0, Wl1, Wl2)
    reshaped = jnp.transpose(mixed, (0, 2, 1))
    return (reshaped, sc)


# speedup vs baseline: 1.0486x; 1.0018x over previous
import dataclasses

import jax
import jax.numpy as jnp
from jax import lax
from jax.experimental import pallas as pl
from jax.experimental.pallas import tpu as pltpu
from jax.experimental.pallas import tpu_sc as plsc

N = 10000
E = 160000
D = 128
DD = 64

NC = 2
NS = 16
NW = NC * NS
GB = 128
EC = E // 2
CBLK = EC // GB
GW_FULL = CBLK // NW + 1
SW_FULL = CBLK // NS + 1

NROWS = 624
NREM = N - 15 * NROWS - NROWS
ZR = 16

NB = 1000
EB = 1000


def _bdot(a, b):
    return jnp.dot(a.astype(jnp.bfloat16), b.astype(jnp.bfloat16),
                   preferred_element_type=jnp.float32)


def _node_body(nf_ref, wsk_ref, wup_ref, wdn_ref, sc_ref, xcat_ref, xdn_ref):
    nf = nf_ref[...]
    sc_ref[...] = _bdot(nf, wsk_ref[...])
    up = _bdot(nf, wup_ref[...])
    dn = _bdot(nf, wdn_ref[...])
    pad = jnp.zeros((nf.shape[0], DD), jnp.float32)
    xcat_ref[...] = jnp.concatenate([up, dn, pad], axis=1)
    xdn_ref[...] = jnp.concatenate([dn, pad], axis=1)


def _node_call(node_feats, w_skip, w_up, w_down):
    return pl.pallas_call(
        _node_body,
        grid=(N // NB,),
        in_specs=[
            pl.BlockSpec((NB, D), lambda i: (i, 0)),
            pl.BlockSpec((D, D), lambda i: (0, 0)),
            pl.BlockSpec((D, D), lambda i: (0, 0)),
            pl.BlockSpec((D, DD), lambda i: (0, 0)),
        ],
        out_specs=[
            pl.BlockSpec((NB, D), lambda i: (i, 0)),
            pl.BlockSpec((NB, 2 * D), lambda i: (i, 0)),
            pl.BlockSpec((NB, D), lambda i: (i, 0)),
        ],
        out_shape=[
            jax.ShapeDtypeStruct((N, D), jnp.float32),
            jax.ShapeDtypeStruct((N, 2 * D), jnp.float32),
            jax.ShapeDtypeStruct((N, D), jnp.float32),
        ],
    )(node_feats, w_skip, w_up, w_down)


def _gather_body(xcat_hbm, xdn_hbm, snd_hbm, rcv_hbm, gs_hbm, gr_hbm,
                 idx_s, idx_r, rows_sa, rows_sb, rows_ra, rows_rb,
                 sem_sa, sem_sb, sem_ra, sem_rb):
    cid = lax.axis_index("c")
    sid = lax.axis_index("s")
    wid = sid * NC + cid
    nblk = jnp.where(wid < CBLK % NW, CBLK // NW + 1, CBLK // NW)

    @pl.loop(0, nblk)
    def _(b):
        off = (wid + b * NW) * GB
        pltpu.sync_copy(snd_hbm.at[pl.ds(off, GB)], idx_s.at[b])
        pltpu.sync_copy(rcv_hbm.at[pl.ds(off, GB)], idx_r.at[b])

    def start(b, rows_s, rows_r, sem_s, sem_r):
        pltpu.async_copy(xcat_hbm.at[idx_s.at[b]], rows_s, sem_s)
        pltpu.async_copy(xdn_hbm.at[idx_r.at[b]], rows_r, sem_r)

    def drain(b, rows_s, rows_r, sem_s, sem_r):
        pltpu.make_async_copy(xcat_hbm.at[idx_s.at[0]], rows_s, sem_s).wait()
        pltpu.make_async_copy(xdn_hbm.at[idx_r.at[0]], rows_r, sem_r).wait()
        off = (wid + b * NW) * GB
        pltpu.sync_copy(rows_s, gs_hbm.at[pl.ds(off, GB)])
        pltpu.sync_copy(rows_r, gr_hbm.at[pl.ds(off, GB)])

    start(0, rows_sa, rows_ra, sem_sa, sem_ra)

    @pl.loop(0, (GW_FULL + 1) // 2)
    def _(i):
        b = 2 * i

        @pl.when(b + 1 < nblk)
        def _():
            start(b + 1, rows_sb, rows_rb, sem_sb, sem_rb)

        @pl.when(b < nblk)
        def _():
            drain(b, rows_sa, rows_ra, sem_sa, sem_ra)

        @pl.when(b + 2 < nblk)
        def _():
            start(b + 2, rows_sa, rows_ra, sem_sa, sem_ra)

        @pl.when(b + 1 < nblk)
        def _():
            drain(b + 1, rows_sb, rows_rb, sem_sb, sem_rb)


def _gather_call(xcat, xdn, sender, receiver):
    mesh = plsc.VectorSubcoreMesh(core_axis_name="c", subcore_axis_name="s")
    f = pl.kernel(
        _gather_body,
        out_type=[
            jax.ShapeDtypeStruct((EC, 2 * D), jnp.float32),
            jax.ShapeDtypeStruct((EC, D), jnp.float32),
        ],
        mesh=mesh,
        scratch_types=[
            pltpu.VMEM((GW_FULL, GB), jnp.int32),
            pltpu.VMEM((GW_FULL, GB), jnp.int32),
            pltpu.VMEM((GB, 2 * D), jnp.float32),
            pltpu.VMEM((GB, 2 * D), jnp.float32),
            pltpu.VMEM((GB, D), jnp.float32),
            pltpu.VMEM((GB, D), jnp.float32),
            pltpu.SemaphoreType.DMA,
            pltpu.SemaphoreType.DMA,
            pltpu.SemaphoreType.DMA,
            pltpu.SemaphoreType.DMA,
        ],
    )
    return f(xcat, xdn, sender, receiver)


def _mlp_body(ef_ref, ea_ref, gs_ref, gr_ref,
              wm0_ref, wm1_ref, wm2_ref, wm3_ref, y_ref):
    gs = gs_ref[...]
    xs = gs[:, :D]
    aug = jnp.concatenate([ef_ref[...], gs[:, D:D + DD], gr_ref[...][:, :DD]],
                          axis=1)
    h = jax.nn.silu(_bdot(aug, wm0_ref[...]))
    h = jax.nn.silu(_bdot(h, wm1_ref[...]))
    h = jax.nn.silu(_bdot(h, wm2_ref[...]))
    tpw = _bdot(h, wm3_ref[...])
    ea = ea_ref[...]
    y0 = xs * tpw[:, :D]
    y1 = xs * tpw[:, D:2 * D]
    y2 = xs * tpw[:, 2 * D:]
    comps = [y0 * ea[:, 0:1]]
    comps += [y1 * ea[:, k:k + 1] for k in range(1, 4)]
    comps += [y2 * ea[:, k:k + 1] for k in range(4, 9)]
    y_ref[...] = jnp.stack(comps)


def _mlp_call(edge_feats, edge_attrs, gs, gr, wm0, wm1, wm2, wm3, chunk):
    co = chunk * (EC // EB)
    return pl.pallas_call(
        _mlp_body,
        grid=(EC // EB,),
        in_specs=[
            pl.BlockSpec((EB, 8), lambda i: (i + co, 0)),
            pl.BlockSpec((EB, 9), lambda i: (i + co, 0)),
            pl.BlockSpec((EB, 2 * D), lambda i: (i, 0)),
            pl.BlockSpec((EB, D), lambda i: (i, 0)),
            pl.BlockSpec((8 + 2 * DD, 256), lambda i: (0, 0)),
            pl.BlockSpec((256, 256), lambda i: (0, 0)),
            pl.BlockSpec((256, 256), lambda i: (0, 0)),
            pl.BlockSpec((256, 3 * D), lambda i: (0, 0)),
        ],
        out_specs=pl.BlockSpec((9, EB, D), lambda i: (0, i, 0)),
        out_shape=jax.ShapeDtypeStruct((9, EC, D), jnp.float32),
    )(edge_feats, edge_attrs, gs, gr, wm0, wm1, wm2, wm3)


def _scatter_body(y0_hbm, y1_hbm, rcv_hbm, parts_hbm,
                  acc, ybuf_a, ybuf_b, idxall, zbuf, sem_a, sem_b):
    cid = lax.axis_index("c")
    sid = lax.axis_index("s")
    nblk = jnp.where(sid < CBLK % NS, CBLK // NS + 1, CBLK // NS)
    zero16 = jnp.zeros((16,), jnp.float32)

    @pl.loop(0, ZR)
    def _(r):
        for c in range(D // 16):
            zbuf[r, pl.ds(c * 16, 16)] = zero16

    @pl.loop(0, nblk)
    def _(b):
        off = cid * EC + (sid + b * NS) * GB
        pltpu.sync_copy(rcv_hbm.at[pl.ds(off, GB)], idxall.at[b])

    def passes(y_hbm):
        def start(j, b, buf, sem):
            pltpu.async_copy(y_hbm.at[j, pl.ds((sid + b * NS) * GB, GB)],
                             buf, sem)

        def wait(j, buf, sem):
            pltpu.make_async_copy(y_hbm.at[j, pl.ds(0, GB)], buf, sem).wait()

        def scat(b, buf):
            pltpu.sync_copy(buf, acc.at[idxall.at[b]], add=True)

        @pl.loop(0, 9)
        def _(j):
            @pl.loop(0, NROWS // ZR)
            def _(k):
                pltpu.sync_copy(zbuf, acc.at[pl.ds(sid * NROWS + k * ZR, ZR)])

            @pl.when(sid == NS - 1)
            def _():
                pltpu.sync_copy(zbuf, acc.at[pl.ds(N - NREM, NREM)])
            plsc.subcore_barrier()

            start(j, 0, ybuf_a, sem_a)

            @pl.loop(0, (SW_FULL + 1) // 2)
            def _(i):
                b = 2 * i

                @pl.when(b + 1 < nblk)
                def _():
                    start(j, b + 1, ybuf_b, sem_b)

                @pl.when(b < nblk)
                def _():
                    wait(j, ybuf_a, sem_a)
                    scat(b, ybuf_a)

                @pl.when(b + 2 < nblk)
                def _():
                    start(j, b + 2, ybuf_a, sem_a)

                @pl.when(b + 1 < nblk)
                def _():
                    wait(j, ybuf_b, sem_b)
                    scat(b + 1, ybuf_b)

            plsc.subcore_barrier()
            pltpu.sync_copy(acc.at[pl.ds(sid * NROWS, NROWS)],
                            parts_hbm.at[cid, j, pl.ds(sid * NROWS, NROWS)])

            @pl.when(sid == NS - 1)
            def _():
                pltpu.sync_copy(acc.at[pl.ds(N - NREM, NREM)],
                                parts_hbm.at[cid, j, pl.ds(N - NREM, NREM)])
            plsc.subcore_barrier()

    @pl.when(cid == 0)
    def _():
        passes(y0_hbm)

    @pl.when(cid == 1)
    def _():
        passes(y1_hbm)


def _scatter_call(y0, y1, receiver):
    mesh = plsc.VectorSubcoreMesh(core_axis_name="c", subcore_axis_name="s")
    cp = pltpu.CompilerParams()
    if "needs_layout_passes" in pltpu.CompilerParams.__dataclass_fields__:
        cp = dataclasses.replace(cp, needs_layout_passes=False)
    f = pl.kernel(
        _scatter_body,
        out_type=jax.ShapeDtypeStruct((NC, 9, N, D), jnp.float32),
        mesh=mesh,
        compiler_params=cp,
        scratch_types=[
            pltpu.VMEM_SHARED((N, D), jnp.float32),
            pltpu.VMEM((GB, D), jnp.float32),
            pltpu.VMEM((GB, D), jnp.float32),
            pltpu.VMEM((SW_FULL, GB), jnp.int32),
            pltpu.VMEM((ZR, D), jnp.float32),
            pltpu.SemaphoreType.DMA,
            pltpu.SemaphoreType.DMA,
        ],
    )
    return f(y0, y1, receiver)


def _mix_body3(parts_ref, wl0_ref, wl1_ref, wl2_ref, out_ref):
    inv = jnp.float32(1.0 / 16.0)
    p = parts_ref[...]
    outs = []
    for j in range(9):
        msg = p[0, j] + p[1, j]
        if j == 0:
            w = wl0_ref[...]
        elif j < 4:
            w = wl1_ref[...]
        else:
            w = wl2_ref[...]
        outs.append(_bdot(msg, w) * inv)
    out_ref[...] = jnp.stack(outs, axis=1)


def _mix_call3(parts, wl0, wl1, wl2):
    return pl.pallas_call(
        _mix_body3,
        grid=(N // NB,),
        in_specs=[
            pl.BlockSpec((NC, 9, NB, D), lambda i: (0, 0, i, 0)),
            pl.BlockSpec((D, D), lambda i: (0, 0)),
            pl.BlockSpec((D, D), lambda i: (0, 0)),
            pl.BlockSpec((D, D), lambda i: (0, 0)),
        ],
        out_specs=pl.BlockSpec((NB, 9, D), lambda i: (i, 0, 0)),
        out_shape=jax.ShapeDtypeStruct((N, 9, D), jnp.float32),
    )(parts, wl0, wl1, wl2)


def kernel(node_attrs, node_feats, edge_attrs, edge_feats, edge_index,
           W_up, W_down, Wm0, Wm1, Wm2, Wm3, Wl0, Wl1, Wl2, W_skip):
    sender = edge_index[0]
    receiver = edge_index[1]

    sc, xcat, xdn = _node_call(node_feats, W_skip, W_up, W_down)
    gs0, gr0 = _gather_call(xcat, xdn, sender[:EC], receiver[:EC])
    gs1, gr1 = _gather_call(xcat, xdn, sender[EC:], receiver[EC:])
    y0 = _mlp_call(edge_feats, edge_attrs, gs0, gr0, Wm0, Wm1, Wm2, Wm3, 0)
    y1 = _mlp_call(edge_feats, edge_attrs, gs1, gr1, Wm0, Wm1, Wm2, Wm3, 1)
    parts = _scatter_call(y0, y1, receiver)
    mixed = _mix_call3(parts, Wl0, Wl1, Wl2)
    reshaped = jnp.transpose(mixed, (0, 2, 1))
    return (reshaped, sc)
